# Initial kernel scaffold; baseline (speedup 1.0000x reference)
import jax
import jax.numpy as jnp
from jax.experimental import pallas as pl


def kernel(x, edge_index, edge_attr, batch, num_graphs, W_l1, W_r1, W_e1, att1, b1, W_l2, W_r2, W_e2, att2, b2, W_ih, W_hh, b_ih, b_hh, lin1_W, lin1_b, lin2_W, lin2_b):
    raise NotImplementedError("write your pallas kernel here")


# speedup vs baseline: 14.1530x; 14.1530x over previous
import functools

import jax
import jax.numpy as jnp
from jax import lax
from jax.experimental import pallas as pl
from jax.experimental.pallas import tpu as pltpu
from jax.experimental.pallas import tpu_sc as plsc

N = 10000
E = 320000
DF = 128
DE = 16
HC = 64
NG = 20

NC = 2
NS = 16
LANES = 16
NW = NC * NS
EPW = E // NW
RU = 80
NU = N // RU
_D1 = 3 * HC
_ROW = HC + LANES

_mesh = plsc.VectorSubcoreMesh(core_axis_name="c", subcore_axis_name="s")


def _n_units(s):
    return (NU - s + NS - 1) // NS


_GDN = lax.GatherDimensionNumbers(
    offset_dims=(), collapsed_slice_dims=(0,), start_index_map=(0,))


def _lperm(v, idx):
    return lax.gather(v, idx[:, None], dimension_numbers=_GDN, slice_sizes=(1,),
                      mode=lax.GatherScatterMode.PROMISE_IN_BOUNDS)


def _hsum(v, iot):
    for sh in (8, 4, 2, 1):
        v = v + _lperm(v, iot ^ sh)
    return v


def _zero_spmem(acc, zbuf, s, row_w):
    zv = jnp.zeros((LANES,), jnp.float32)

    def zrow(r, carry):
        for v in range(row_w // LANES):
            zbuf[r, pl.ds(LANES * v, LANES)] = zv
        return carry

    lax.fori_loop(0, RU, zrow, 0)

    def unit(j, carry):
        pltpu.sync_copy(zbuf, acc.at[pl.ds((s + NS * j) * RU, RU)])
        return carry

    lax.fori_loop(0, _n_units(s), unit, 0)


_B0 = 80


@functools.partial(
    pl.kernel,
    out_type=jax.ShapeDtypeStruct((NC, N, 32), jnp.float32),
    mesh=_mesh,
    scratch_types=[
        pltpu.VMEM((_B0,), jnp.int32),
        pltpu.VMEM((_B0, DE), jnp.float32),
        pltpu.VMEM((_B0, 32), jnp.float32),
        pltpu.VMEM((RU, 32), jnp.float32),
        pltpu.VMEM_SHARED((N, 32), jnp.float32),
    ],
    compiler_params=pltpu.CompilerParams(use_tc_tiling_on_sc=False),
)
def _sc_hist(tgt_hbm, attr_hbm, out_hbm, tgt_v, attr_v, row_v, zbuf, acc):
    c = lax.axis_index("c")
    s = lax.axis_index("s")
    wid = s * NC + c
    _zero_spmem(acc, zbuf, s, 32)
    plsc.subcore_barrier()

    iot = lax.iota(jnp.int32, LANES)
    cntvec = jnp.where(iot == 0, 1.0, 0.0).astype(jnp.float32)

    def chunk(i, carry):
        base = wid * EPW + i * _B0
        pltpu.sync_copy(tgt_hbm.at[pl.ds(base, _B0)], tgt_v)
        pltpu.sync_copy(attr_hbm.at[pl.ds(base, _B0)], attr_v)

        def edge(e, cc):
            row_v[e, pl.ds(0, LANES)] = attr_v[e, pl.ds(0, LANES)]
            row_v[e, pl.ds(LANES, LANES)] = cntvec
            return cc

        lax.fori_loop(0, _B0, edge, 0)
        pltpu.sync_copy(row_v, acc.at[tgt_v], add=True)
        return carry

    lax.fori_loop(0, EPW // _B0, chunk, 0)
    plsc.subcore_barrier()

    def unit(j, carry):
        r0 = (s + NS * j) * RU
        pltpu.sync_copy(acc.at[pl.ds(r0, RU)], out_hbm.at[c, pl.ds(r0, RU)])
        return carry

    lax.fori_loop(0, _n_units(s), unit, 0)


_BE = 80


def _make_gat_pass(H):
    @functools.partial(
        pl.kernel,
        out_type=jax.ShapeDtypeStruct((H, NC, N, _ROW), jnp.float32),
        mesh=_mesh,
        scratch_types=[
            pltpu.VMEM((_BE,), jnp.int32),
            pltpu.VMEM((_BE,), jnp.int32),
            pltpu.VMEM((_BE, HC), jnp.float32),
            pltpu.VMEM((_BE, HC), jnp.float32),
            pltpu.VMEM((_BE, HC), jnp.float32),
            pltpu.VMEM((_BE, _ROW), jnp.float32),
            pltpu.VMEM((RU, _ROW), jnp.float32),
            pltpu.VMEM((H * HC,), jnp.float32),
            pltpu.VMEM_SHARED((N, _ROW), jnp.float32),
            pltpu.SemaphoreType.DMA,
            pltpu.SemaphoreType.DMA,
            pltpu.SemaphoreType.DMA,
        ],
        compiler_params=pltpu.CompilerParams(use_tc_tiling_on_sc=False),
    )
    def gat_pass(src_hbm, tgt_hbm, xl_hbm, xr_hbm, ea_hbm, att_hbm, out_hbm,
                 src_v, tgt_v, xl_v, xr_v, ea_v, row_v, zbuf, att_v, acc,
                 sem1, sem2, sem3):
        c = lax.axis_index("c")
        s = lax.axis_index("s")
        wid = s * NC + c
        pltpu.sync_copy(att_hbm, att_v)
        iot = lax.iota(jnp.int32, LANES)

        for h in range(H):
            _zero_spmem(acc, zbuf, s, _ROW)
            plsc.subcore_barrier()
            atts = [att_v[pl.ds(h * HC + LANES * v, LANES)]
                    for v in range(HC // LANES)]

            def chunk(i, carry):
                base = wid * EPW + i * _BE
                pltpu.sync_copy(src_hbm.at[pl.ds(base, _BE)], src_v)
                pltpu.sync_copy(tgt_hbm.at[pl.ds(base, _BE)], tgt_v)
                cp1 = pltpu.async_copy(xl_hbm.at[h].at[src_v], xl_v, sem1)
                cp2 = pltpu.async_copy(xr_hbm.at[h].at[tgt_v], xr_v, sem2)
                cp3 = pltpu.async_copy(ea_hbm.at[h].at[pl.ds(base, _BE)],
                                       ea_v, sem3)
                cp1.wait()
                cp2.wait()
                cp3.wait()

                def edge(e, cc):
                    xls = [xl_v[e, pl.ds(LANES * v, LANES)] for v in range(4)]
                    p = None
                    for v in range(4):
                        z = xls[v] + xr_v[e, pl.ds(LANES * v, LANES)] \
                            + ea_v[e, pl.ds(LANES * v, LANES)]
                        z = jnp.where(z > 0.0, z, 0.2 * z)
                        t = z * atts[v]
                        p = t if p is None else p + t
                    exv = jnp.exp(_hsum(p, iot))
                    for v in range(4):
                        row_v[e, pl.ds(LANES * v, LANES)] = xls[v] * exv
                    row_v[e, pl.ds(HC, LANES)] = jnp.where(iot == 0, exv, 0.0)
                    return cc

                lax.fori_loop(0, _BE, edge, 0)
                pltpu.sync_copy(row_v, acc.at[tgt_v], add=True)
                return carry

            lax.fori_loop(0, EPW // _BE, chunk, 0)
            plsc.subcore_barrier()

            def unit(j, carry):
                r0 = (s + NS * j) * RU
                pltpu.sync_copy(acc.at[pl.ds(r0, RU)],
                                out_hbm.at[h].at[c].at[pl.ds(r0, RU)])
                return carry

            lax.fori_loop(0, _n_units(s), unit, 0)

    return gat_pass


_sc_gat3 = _make_gat_pass(3)
_sc_gat1 = _make_gat_pass(1)


_NBLK = 1000
_EBLK = 4000


def _tc_nodes_body(x_ref, wl_ref, wr_ref, xl_ref, xr_ref):
    x = x_ref[...]
    xl_ref[...] = jnp.dot(x, wl_ref[...],
                          preferred_element_type=jnp.float32).reshape(
                              -1, 3, HC).swapaxes(0, 1)
    xr_ref[...] = jnp.dot(x, wr_ref[...],
                          preferred_element_type=jnp.float32).reshape(
                              -1, 3, HC).swapaxes(0, 1)


def _tc_ea_body(attr_ref, we1_ref, we2_ref, ea1_ref, ea2_ref):
    a = attr_ref[...]
    ea1_ref[...] = jnp.dot(a, we1_ref[...],
                           preferred_element_type=jnp.float32).reshape(
                               -1, 3, HC).swapaxes(0, 1)
    ea2_ref[...] = jnp.dot(a, we2_ref[...], preferred_element_type=jnp.float32)


def _tc_asm1_body(acc_ref, attr_ref, xl_ref, xr_ref, we1_ref, att1_ref, b1_ref,
                  wl2_ref, wr2_ref, we2_ref, xl2_ref, xr2_ref, sea2_ref):
    attr = attr_ref[0] + attr_ref[1]
    cnt = jnp.clip(attr[:, DE:DE + 1], 1.0, None)
    loop_attr = attr[:, :DE] / cnt
    sea = jnp.dot(loop_attr, we1_ref[...], preferred_element_type=jnp.float32)
    outs = []
    for h in range(3):
        xlh = xl_ref[h]
        zh = xlh + xr_ref[h] + sea[:, HC * h:HC * (h + 1)]
        zh = jnp.where(zh > 0.0, zh, 0.2 * zh)
        alpha = jnp.sum(zh * att1_ref[h][None, :], axis=1, keepdims=True)
        ex = jnp.exp(alpha)
        accs = acc_ref[h, 0] + acc_ref[h, 1]
        num = accs[:, :HC] + ex * xlh
        den = accs[:, HC:HC + 1] + ex
        outs.append(num / den)
    hh = jnp.concatenate(outs, axis=1) + b1_ref[...]
    hh = jnp.where(hh > 0.0, hh, jnp.exp(jnp.minimum(hh, 0.0)) - 1.0)
    xl2_ref[...] = jnp.dot(hh, wl2_ref[...], preferred_element_type=jnp.float32)
    xr2_ref[...] = jnp.dot(hh, wr2_ref[...], preferred_element_type=jnp.float32)
    sea2_ref[...] = jnp.dot(loop_attr, we2_ref[...],
                            preferred_element_type=jnp.float32)


def _tc_final_body(acc_ref, xl2_ref, xr2_ref, sea2_ref, att2_ref, b2_ref,
                   batch_ref, wih_ref, whh_ref, bih_ref, bhh_ref,
                   l1w_ref, l1b_ref, l2w_ref, l2b_ref, out_ref, fr_ref):
    acc = acc_ref[0, 0] + acc_ref[0, 1]
    xl2 = xl2_ref[...]
    z = xl2 + xr2_ref[...] + sea2_ref[...]
    z = jnp.where(z > 0.0, z, 0.2 * z)
    alpha = jnp.sum(z * att2_ref[...], axis=1, keepdims=True)
    ex = jnp.exp(alpha)
    num = acc[:, :HC] + ex * xl2
    den = acc[:, HC:HC + 1] + ex
    h2 = num / den + b2_ref[...]
    h2 = jnp.where(h2 > 0.0, h2, jnp.exp(jnp.minimum(h2, 0.0)) - 1.0)

    gids = lax.broadcasted_iota(jnp.int32, (NG, N), 0)
    mask = (gids == batch_ref[...]).astype(jnp.float32)
    gsum = jnp.dot(mask, h2, preferred_element_type=jnp.float32)
    gcnt = jnp.clip(jnp.sum(mask, axis=1, keepdims=True), 1.0, None)
    fr_ref[...] = gsum / gcnt

    wih = wih_ref[...]
    whh = whh_ref[...]
    bsum = bih_ref[...] + bhh_ref[...]

    def step(t, carry):
        hh, cc = carry
        xt = fr_ref[pl.ds(t, 1), :]
        g = jnp.dot(xt, wih, preferred_element_type=jnp.float32) \
            + jnp.dot(hh, whh, preferred_element_type=jnp.float32) + bsum
        gi = jax.nn.sigmoid(g[:, :HC])
        gf = jax.nn.sigmoid(g[:, HC:2 * HC])
        gg = jnp.tanh(g[:, 2 * HC:3 * HC])
        go = jax.nn.sigmoid(g[:, 3 * HC:])
        c2 = gf * cc + gi * gg
        return (go * jnp.tanh(c2), c2)

    zed = jnp.zeros((1, HC), jnp.float32)
    hlast, _ = lax.fori_loop(0, NG, step, (zed, zed))
    o = jnp.dot(hlast, l1w_ref[...], preferred_element_type=jnp.float32) \
        + l1b_ref[...]
    o = jnp.maximum(o, 0.0)
    out_ref[...] = jnp.dot(o, l2w_ref[...],
                           preferred_element_type=jnp.float32) + l2b_ref[...]


def kernel(x, edge_index, edge_attr, batch, num_graphs, W_l1, W_r1, W_e1, att1,
           b1, W_l2, W_r2, W_e2, att2, b2, W_ih, W_hh, b_ih, b_hh, lin1_W,
           lin1_b, lin2_W, lin2_b):
    f32 = jnp.float32
    src = edge_index[0]
    tgt = edge_index[1]

    xl1, xr1 = pl.pallas_call(
        _tc_nodes_body,
        grid=(N // _NBLK,),
        in_specs=[
            pl.BlockSpec((_NBLK, DF), lambda i: (i, 0)),
            pl.BlockSpec((DF, _D1), lambda i: (0, 0)),
            pl.BlockSpec((DF, _D1), lambda i: (0, 0)),
        ],
        out_specs=[
            pl.BlockSpec((3, _NBLK, HC), lambda i: (0, i, 0)),
            pl.BlockSpec((3, _NBLK, HC), lambda i: (0, i, 0)),
        ],
        out_shape=[
            jax.ShapeDtypeStruct((3, N, HC), f32),
            jax.ShapeDtypeStruct((3, N, HC), f32),
        ],
    )(x, W_l1, W_r1)

    ea1, ea2b = pl.pallas_call(
        _tc_ea_body,
        grid=(E // _EBLK,),
        in_specs=[
            pl.BlockSpec((_EBLK, DE), lambda i: (i, 0)),
            pl.BlockSpec((DE, _D1), lambda i: (0, 0)),
            pl.BlockSpec((DE, HC), lambda i: (0, 0)),
        ],
        out_specs=[
            pl.BlockSpec((3, _EBLK, HC), lambda i: (0, i, 0)),
            pl.BlockSpec((_EBLK, HC), lambda i: (i, 0)),
        ],
        out_shape=[
            jax.ShapeDtypeStruct((3, E, HC), f32),
            jax.ShapeDtypeStruct((E, HC), f32),
        ],
    )(edge_attr, W_e1, W_e2)

    attr_p = _sc_hist(tgt, edge_attr)
    acc1_p = _sc_gat3(src, tgt, xl1, xr1, ea1, att1.reshape(_D1))

    xl2, xr2, sea2 = pl.pallas_call(
        _tc_asm1_body,
        grid=(N // _NBLK,),
        in_specs=[
            pl.BlockSpec((3, NC, _NBLK, _ROW), lambda i: (0, 0, i, 0)),
            pl.BlockSpec((NC, _NBLK, 32), lambda i: (0, i, 0)),
            pl.BlockSpec((3, _NBLK, HC), lambda i: (0, i, 0)),
            pl.BlockSpec((3, _NBLK, HC), lambda i: (0, i, 0)),
            pl.BlockSpec((DE, _D1), lambda i: (0, 0)),
            pl.BlockSpec((3, HC), lambda i: (0, 0)),
            pl.BlockSpec((1, _D1), lambda i: (0, 0)),
            pl.BlockSpec((_D1, HC), lambda i: (0, 0)),
            pl.BlockSpec((_D1, HC), lambda i: (0, 0)),
            pl.BlockSpec((DE, HC), lambda i: (0, 0)),
        ],
        out_specs=[
            pl.BlockSpec((_NBLK, HC), lambda i: (i, 0)),
            pl.BlockSpec((_NBLK, HC), lambda i: (i, 0)),
            pl.BlockSpec((_NBLK, HC), lambda i: (i, 0)),
        ],
        out_shape=[
            jax.ShapeDtypeStruct((N, HC), f32),
            jax.ShapeDtypeStruct((N, HC), f32),
            jax.ShapeDtypeStruct((N, HC), f32),
        ],
    )(acc1_p, attr_p, xl1, xr1, W_e1, att1, b1.reshape(1, _D1),
      W_l2, W_r2, W_e2)

    acc2_p = _sc_gat1(src, tgt, xl2.reshape(1, N, HC), xr2.reshape(1, N, HC),
                      ea2b.reshape(1, E, HC), att2.reshape(HC))

    out = pl.pallas_call(
        _tc_final_body,
        out_shape=jax.ShapeDtypeStruct((1, 1), f32),
        scratch_shapes=[pltpu.VMEM((NG, HC), f32)],
    )(acc2_p, xl2, xr2, sea2, att2, b2.reshape(1, HC), batch.reshape(1, N),
      W_ih.T, W_hh.T, b_ih.reshape(1, 4 * HC), b_hh.reshape(1, 4 * HC),
      lin1_W, lin1_b.reshape(1, HC // 2), lin2_W, lin2_b.reshape(1, 1))
    return out


# speedup vs baseline: 14.5060x; 1.0249x over previous
import functools

import jax
import jax.numpy as jnp
from jax import lax
from jax.experimental import pallas as pl
from jax.experimental.pallas import tpu as pltpu
from jax.experimental.pallas import tpu_sc as plsc

N = 10000
E = 320000
DF = 128
DE = 16
HC = 64
NG = 20

NC = 2
NS = 16
LANES = 16
NW = NC * NS
EPW = E // NW
RU = 80
NU = N // RU
_D1 = 3 * HC
_ROW = HC + LANES

_mesh = plsc.VectorSubcoreMesh(core_axis_name="c", subcore_axis_name="s")


def _n_units(s):
    return (NU - s + NS - 1) // NS


_GDN = lax.GatherDimensionNumbers(
    offset_dims=(), collapsed_slice_dims=(0,), start_index_map=(0,))


def _lperm(v, idx):
    return lax.gather(v, idx[:, None], dimension_numbers=_GDN, slice_sizes=(1,),
                      mode=lax.GatherScatterMode.PROMISE_IN_BOUNDS)


def _hsum(v, iot):
    for sh in (8, 4, 2, 1):
        v = v + _lperm(v, iot ^ sh)
    return v


def _sc_exp(a):
    y = jnp.clip(a * 1.4426950408889634, -126.0, 126.0)
    n = (y + 512.5).astype(jnp.int32) - 512
    t = (y - n.astype(jnp.float32)) * 0.6931471805599453
    p = 1.0 + t * (1.0 + t * (0.5 + t * (
        0.16666666666666666 + t * (0.041666666666666664 + t * 0.008333333333333333))))
    scale = lax.bitcast_convert_type(
        lax.shift_left(n + 127, 23), jnp.float32)
    return p * scale


def _zero_spmem(acc, zbuf, s, row_w):
    zv = jnp.zeros((LANES,), jnp.float32)

    def zrow(r, carry):
        for v in range(row_w // LANES):
            zbuf[r, pl.ds(LANES * v, LANES)] = zv
        return carry

    lax.fori_loop(0, RU, zrow, 0)

    def unit(j, carry):
        pltpu.sync_copy(zbuf, acc.at[pl.ds((s + NS * j) * RU, RU)])
        return carry

    lax.fori_loop(0, _n_units(s), unit, 0)


_B0 = 80


@functools.partial(
    pl.kernel,
    out_type=jax.ShapeDtypeStruct((NC, N, 32), jnp.float32),
    mesh=_mesh,
    scratch_types=[
        pltpu.VMEM((_B0,), jnp.int32),
        pltpu.VMEM((_B0, DE), jnp.float32),
        pltpu.VMEM((_B0, 32), jnp.float32),
        pltpu.VMEM((RU, 32), jnp.float32),
        pltpu.VMEM_SHARED((N, 32), jnp.float32),
    ],
    compiler_params=pltpu.CompilerParams(use_tc_tiling_on_sc=False),
)
def _sc_hist(tgt_hbm, attr_hbm, out_hbm, tgt_v, attr_v, row_v, zbuf, acc):
    c = lax.axis_index("c")
    s = lax.axis_index("s")
    wid = s * NC + c
    _zero_spmem(acc, zbuf, s, 32)
    plsc.subcore_barrier()

    iot = lax.iota(jnp.int32, LANES)
    cntvec = jnp.where(iot == 0, 1.0, 0.0).astype(jnp.float32)

    def chunk(i, carry):
        base = wid * EPW + i * _B0
        pltpu.sync_copy(tgt_hbm.at[pl.ds(base, _B0)], tgt_v)
        pltpu.sync_copy(attr_hbm.at[pl.ds(base, _B0)], attr_v)

        def edge(e, cc):
            row_v[e, pl.ds(0, LANES)] = attr_v[e, pl.ds(0, LANES)]
            row_v[e, pl.ds(LANES, LANES)] = cntvec
            return cc

        lax.fori_loop(0, _B0, edge, 0)
        pltpu.sync_copy(row_v, acc.at[tgt_v], add=True)
        return carry

    lax.fori_loop(0, EPW // _B0, chunk, 0)
    plsc.subcore_barrier()

    def unit(j, carry):
        r0 = (s + NS * j) * RU
        pltpu.sync_copy(acc.at[pl.ds(r0, RU)], out_hbm.at[c, pl.ds(r0, RU)])
        return carry

    lax.fori_loop(0, _n_units(s), unit, 0)


_BE = 80
_NCH = EPW // _BE


def _make_gat_pass(H):
    @functools.partial(
        pl.kernel,
        out_type=jax.ShapeDtypeStruct((H, NC, N, _ROW), jnp.float32),
        mesh=_mesh,
        scratch_types=[
            pltpu.VMEM((EPW,), jnp.int32),
            pltpu.VMEM((EPW,), jnp.int32),
            [pltpu.VMEM((_BE, HC), jnp.float32) for _ in range(2)],
            [pltpu.VMEM((_BE, HC), jnp.float32) for _ in range(2)],
            [pltpu.VMEM((_BE, HC), jnp.float32) for _ in range(2)],
            [pltpu.VMEM((_BE, _ROW), jnp.float32) for _ in range(2)],
            [pltpu.VMEM((_BE,), jnp.int32) for _ in range(2)],
            pltpu.VMEM((RU, _ROW), jnp.float32),
            pltpu.VMEM((H * HC,), jnp.float32),
            pltpu.VMEM_SHARED((N, _ROW), jnp.float32),
            [pltpu.SemaphoreType.DMA for _ in range(2)],
            [pltpu.SemaphoreType.DMA for _ in range(2)],
        ],
        compiler_params=pltpu.CompilerParams(use_tc_tiling_on_sc=False),
    )
    def gat_pass(src_hbm, tgt_hbm, xl_hbm, xr_hbm, ea_hbm, att_hbm, out_hbm,
                 src_all, tgt_all, xl_v, xr_v, ea_v, row_v, tgt_v, zbuf,
                 att_v, acc, gsem, ssem):
        c = lax.axis_index("c")
        s = lax.axis_index("s")
        wid = s * NC + c
        ebase = wid * EPW
        pltpu.sync_copy(att_hbm, att_v)
        pltpu.sync_copy(src_hbm.at[pl.ds(ebase, EPW)], src_all)
        pltpu.sync_copy(tgt_hbm.at[pl.ds(ebase, EPW)], tgt_all)
        iot = lax.iota(jnp.int32, LANES)

        for h in range(H):
            _zero_spmem(acc, zbuf, s, _ROW)
            plsc.subcore_barrier()
            atts = [att_v[pl.ds(h * HC + LANES * v, LANES)]
                    for v in range(HC // LANES)]

            def issue_gathers(ci, b):
                pltpu.async_copy(
                    xl_hbm.at[h].at[src_all.at[pl.ds(ci * _BE, _BE)]],
                    xl_v[b], gsem[b])
                pltpu.async_copy(
                    xr_hbm.at[h].at[tgt_all.at[pl.ds(ci * _BE, _BE)]],
                    xr_v[b], gsem[b])
                pltpu.async_copy(
                    ea_hbm.at[h].at[pl.ds(ebase + ci * _BE, _BE)],
                    ea_v[b], gsem[b])

            def wait_gathers(b):
                for _ in range(3):
                    pltpu.make_async_copy(
                        ea_hbm.at[h].at[pl.ds(0, _BE)], xl_v[b],
                        gsem[b]).wait()

            def wait_scatter(b):
                pass

            def compute_scatter(ci, b):
                xlb, xrb, eab, rowb, tgtb = (xl_v[b], xr_v[b], ea_v[b],
                                             row_v[b], tgt_v[b])

                def edge(e, cc):
                    xls = [xlb[e, pl.ds(LANES * v, LANES)] for v in range(4)]
                    p = None
                    for v in range(4):
                        z = xls[v] + xrb[e, pl.ds(LANES * v, LANES)] \
                            + eab[e, pl.ds(LANES * v, LANES)]
                        z = jnp.where(z > 0.0, z, 0.2 * z)
                        t = z * atts[v]
                        p = t if p is None else p + t
                    exv = _sc_exp(_hsum(p, iot))
                    for v in range(4):
                        rowb[e, pl.ds(LANES * v, LANES)] = xls[v] * exv
                    rowb[e, pl.ds(HC, LANES)] = jnp.where(iot == 0, exv, 0.0)
                    return cc

                lax.fori_loop(0, _BE, edge, 0)
                pltpu.sync_copy(tgt_hbm.at[pl.ds(ebase + ci * _BE, _BE)], tgtb)
                pltpu.sync_copy(rowb, acc.at[tgtb], add=True)

            issue_gathers(0, 0)

            def pair(j, carry):
                c0 = 2 * j
                issue_gathers(c0 + 1, 1)
                wait_gathers(0)

                @pl.when(j > 0)
                def _():
                    wait_scatter(0)

                compute_scatter(c0, 0)
                issue_gathers(c0 + 2, 0)
                wait_gathers(1)

                @pl.when(j > 0)
                def _():
                    wait_scatter(1)

                compute_scatter(c0 + 1, 1)
                return carry

            lax.fori_loop(0, (_NCH - 1) // 2, pair, 0)
            wait_gathers(0)
            wait_scatter(0)
            compute_scatter(_NCH - 1, 0)
            wait_scatter(0)
            wait_scatter(1)
            plsc.subcore_barrier()

            def unit(j, carry):
                r0 = (s + NS * j) * RU
                pltpu.sync_copy(acc.at[pl.ds(r0, RU)],
                                out_hbm.at[h].at[c].at[pl.ds(r0, RU)])
                return carry

            lax.fori_loop(0, _n_units(s), unit, 0)

    return gat_pass


_sc_gat3 = _make_gat_pass(3)
_sc_gat1 = _make_gat_pass(1)


_NBLK = 1000
_EBLK = 4000


def _tc_nodes_body(x_ref, wl_ref, wr_ref, xl_ref, xr_ref):
    x = x_ref[...].astype(jnp.bfloat16)
    xl_ref[...] = jnp.dot(x, wl_ref[...].astype(jnp.bfloat16),
                          preferred_element_type=jnp.float32).reshape(
                              -1, 3, HC).swapaxes(0, 1)
    xr_ref[...] = jnp.dot(x, wr_ref[...].astype(jnp.bfloat16),
                          preferred_element_type=jnp.float32).reshape(
                              -1, 3, HC).swapaxes(0, 1)


def _tc_ea_body(attr_ref, we1_ref, we2_ref, ea1_ref, ea2_ref):
    a = attr_ref[...].astype(jnp.bfloat16)
    ea1_ref[...] = jnp.dot(a, we1_ref[...].astype(jnp.bfloat16),
                           preferred_element_type=jnp.float32).reshape(
                               -1, 3, HC).swapaxes(0, 1)
    ea2_ref[...] = jnp.dot(a, we2_ref[...].astype(jnp.bfloat16), preferred_element_type=jnp.float32)


def _tc_asm1_body(acc_ref, attr_ref, xl_ref, xr_ref, we1_ref, att1_ref, b1_ref,
                  wl2_ref, wr2_ref, we2_ref, xl2_ref, xr2_ref, sea2_ref):
    attr = attr_ref[0] + attr_ref[1]
    cnt = jnp.clip(attr[:, DE:DE + 1], 1.0, None)
    loop_attr = attr[:, :DE] / cnt
    sea = jnp.dot(loop_attr.astype(jnp.bfloat16), we1_ref[...].astype(jnp.bfloat16), preferred_element_type=jnp.float32)
    outs = []
    for h in range(3):
        xlh = xl_ref[h]
        zh = xlh + xr_ref[h] + sea[:, HC * h:HC * (h + 1)]
        zh = jnp.where(zh > 0.0, zh, 0.2 * zh)
        alpha = jnp.sum(zh * att1_ref[h][None, :], axis=1, keepdims=True)
        ex = jnp.exp(alpha)
        accs = acc_ref[h, 0] + acc_ref[h, 1]
        num = accs[:, :HC] + ex * xlh
        den = accs[:, HC:HC + 1] + ex
        outs.append(num / den)
    hh = jnp.concatenate(outs, axis=1) + b1_ref[...]
    hh = jnp.where(hh > 0.0, hh, jnp.exp(jnp.minimum(hh, 0.0)) - 1.0)
    xl2_ref[...] = jnp.dot(hh.astype(jnp.bfloat16), wl2_ref[...].astype(jnp.bfloat16), preferred_element_type=jnp.float32)
    xr2_ref[...] = jnp.dot(hh.astype(jnp.bfloat16), wr2_ref[...].astype(jnp.bfloat16), preferred_element_type=jnp.float32)
    sea2_ref[...] = jnp.dot(loop_attr.astype(jnp.bfloat16), we2_ref[...].astype(jnp.bfloat16),
                            preferred_element_type=jnp.float32)


def _tc_final_body(acc_ref, xl2_ref, xr2_ref, sea2_ref, att2_ref, b2_ref,
                   batch_ref, wih_ref, whh_ref, bih_ref, bhh_ref,
                   l1w_ref, l1b_ref, l2w_ref, l2b_ref, out_ref, fr_ref):
    acc = acc_ref[0, 0] + acc_ref[0, 1]
    xl2 = xl2_ref[...]
    z = xl2 + xr2_ref[...] + sea2_ref[...]
    z = jnp.where(z > 0.0, z, 0.2 * z)
    alpha = jnp.sum(z * att2_ref[...], axis=1, keepdims=True)
    ex = jnp.exp(alpha)
    num = acc[:, :HC] + ex * xl2
    den = acc[:, HC:HC + 1] + ex
    h2 = num / den + b2_ref[...]
    h2 = jnp.where(h2 > 0.0, h2, jnp.exp(jnp.minimum(h2, 0.0)) - 1.0)

    gids = lax.broadcasted_iota(jnp.int32, (NG, N), 0)
    mask = (gids == batch_ref[...]).astype(jnp.float32)
    gsum = jnp.dot(mask, h2, preferred_element_type=jnp.float32, precision=lax.Precision.HIGHEST)
    gcnt = jnp.clip(jnp.sum(mask, axis=1, keepdims=True), 1.0, None)
    fr_ref[...] = gsum / gcnt

    wih = wih_ref[...].astype(jnp.bfloat16)
    whh = whh_ref[...].astype(jnp.bfloat16)
    bsum = bih_ref[...] + bhh_ref[...]

    def step(t, carry):
        hh, cc = carry
        xt = fr_ref[pl.ds(t, 1), :]
        g = jnp.dot(xt.astype(jnp.bfloat16), wih, preferred_element_type=jnp.float32) \
            + jnp.dot(hh.astype(jnp.bfloat16), whh, preferred_element_type=jnp.float32) + bsum
        gi = jax.nn.sigmoid(g[:, :HC])
        gf = jax.nn.sigmoid(g[:, HC:2 * HC])
        gg = jnp.tanh(g[:, 2 * HC:3 * HC])
        go = jax.nn.sigmoid(g[:, 3 * HC:])
        c2 = gf * cc + gi * gg
        return (go * jnp.tanh(c2), c2)

    zed = jnp.zeros((1, HC), jnp.float32)
    hlast, _ = lax.fori_loop(0, NG, step, (zed, zed))
    o = jnp.dot(hlast.astype(jnp.bfloat16), l1w_ref[...].astype(jnp.bfloat16), preferred_element_type=jnp.float32) \
        + l1b_ref[...]
    o = jnp.maximum(o, 0.0)
    out_ref[...] = jnp.dot(o.astype(jnp.bfloat16), l2w_ref[...].astype(jnp.bfloat16),
                           preferred_element_type=jnp.float32) + l2b_ref[...]


def kernel(x, edge_index, edge_attr, batch, num_graphs, W_l1, W_r1, W_e1, att1,
           b1, W_l2, W_r2, W_e2, att2, b2, W_ih, W_hh, b_ih, b_hh, lin1_W,
           lin1_b, lin2_W, lin2_b):
    f32 = jnp.float32
    src = edge_index[0]
    tgt = edge_index[1]

    xl1, xr1 = pl.pallas_call(
        _tc_nodes_body,
        grid=(N // _NBLK,),
        in_specs=[
            pl.BlockSpec((_NBLK, DF), lambda i: (i, 0)),
            pl.BlockSpec((DF, _D1), lambda i: (0, 0)),
            pl.BlockSpec((DF, _D1), lambda i: (0, 0)),
        ],
        out_specs=[
            pl.BlockSpec((3, _NBLK, HC), lambda i: (0, i, 0)),
            pl.BlockSpec((3, _NBLK, HC), lambda i: (0, i, 0)),
        ],
        out_shape=[
            jax.ShapeDtypeStruct((3, N, HC), f32),
            jax.ShapeDtypeStruct((3, N, HC), f32),
        ],
    )(x, W_l1, W_r1)

    ea1, ea2b = pl.pallas_call(
        _tc_ea_body,
        grid=(E // _EBLK,),
        in_specs=[
            pl.BlockSpec((_EBLK, DE), lambda i: (i, 0)),
            pl.BlockSpec((DE, _D1), lambda i: (0, 0)),
            pl.BlockSpec((DE, HC), lambda i: (0, 0)),
        ],
        out_specs=[
            pl.BlockSpec((3, _EBLK, HC), lambda i: (0, i, 0)),
            pl.BlockSpec((_EBLK, HC), lambda i: (i, 0)),
        ],
        out_shape=[
            jax.ShapeDtypeStruct((3, E, HC), f32),
            jax.ShapeDtypeStruct((E, HC), f32),
        ],
    )(edge_attr, W_e1, W_e2)

    attr_p = _sc_hist(tgt, edge_attr)
    acc1_p = _sc_gat3(src, tgt, xl1, xr1, ea1, att1.reshape(_D1))

    xl2, xr2, sea2 = pl.pallas_call(
        _tc_asm1_body,
        grid=(N // _NBLK,),
        in_specs=[
            pl.BlockSpec((3, NC, _NBLK, _ROW), lambda i: (0, 0, i, 0)),
            pl.BlockSpec((NC, _NBLK, 32), lambda i: (0, i, 0)),
            pl.BlockSpec((3, _NBLK, HC), lambda i: (0, i, 0)),
            pl.BlockSpec((3, _NBLK, HC), lambda i: (0, i, 0)),
            pl.BlockSpec((DE, _D1), lambda i: (0, 0)),
            pl.BlockSpec((3, HC), lambda i: (0, 0)),
            pl.BlockSpec((1, _D1), lambda i: (0, 0)),
            pl.BlockSpec((_D1, HC), lambda i: (0, 0)),
            pl.BlockSpec((_D1, HC), lambda i: (0, 0)),
            pl.BlockSpec((DE, HC), lambda i: (0, 0)),
        ],
        out_specs=[
            pl.BlockSpec((_NBLK, HC), lambda i: (i, 0)),
            pl.BlockSpec((_NBLK, HC), lambda i: (i, 0)),
            pl.BlockSpec((_NBLK, HC), lambda i: (i, 0)),
        ],
        out_shape=[
            jax.ShapeDtypeStruct((N, HC), f32),
            jax.ShapeDtypeStruct((N, HC), f32),
            jax.ShapeDtypeStruct((N, HC), f32),
        ],
    )(acc1_p, attr_p, xl1, xr1, W_e1, att1, b1.reshape(1, _D1),
      W_l2, W_r2, W_e2)

    acc2_p = _sc_gat1(src, tgt, xl2.reshape(1, N, HC), xr2.reshape(1, N, HC),
                      ea2b.reshape(1, E, HC), att2.reshape(HC))

    out = pl.pallas_call(
        _tc_final_body,
        out_shape=jax.ShapeDtypeStruct((1, 1), f32),
        scratch_shapes=[pltpu.VMEM((NG, HC), f32)],
    )(acc2_p, xl2, xr2, sea2, att2, b2.reshape(1, HC), batch.reshape(1, N),
      W_ih.T, W_hh.T, b_ih.reshape(1, 4 * HC), b_hh.reshape(1, 4 * HC),
      lin1_W, lin1_b.reshape(1, HC // 2), lin2_W, lin2_b.reshape(1, 1))
    return out


# speedup vs baseline: 16.1118x; 1.1107x over previous
import functools

import jax
import jax.numpy as jnp
from jax import lax
from jax.experimental import pallas as pl
from jax.experimental.pallas import tpu as pltpu
from jax.experimental.pallas import tpu_sc as plsc

N = 10000
E = 320000
DF = 128
DE = 16
HC = 64
NG = 20

NC = 2
NS = 16
LANES = 16
NW = NC * NS
EPW = E // NW
RU = 80
NU = N // RU
_D1 = 3 * HC
_ROW = HC + LANES

_mesh = plsc.VectorSubcoreMesh(core_axis_name="c", subcore_axis_name="s")


def _n_units(s):
    return (NU - s + NS - 1) // NS


_GDN = lax.GatherDimensionNumbers(
    offset_dims=(), collapsed_slice_dims=(0,), start_index_map=(0,))


def _lperm(v, idx):
    return lax.gather(v, idx[:, None], dimension_numbers=_GDN, slice_sizes=(1,),
                      mode=lax.GatherScatterMode.PROMISE_IN_BOUNDS)


def _hsum(v, iot):
    for sh in (8, 4, 2, 1):
        v = v + _lperm(v, iot ^ sh)
    return v


def _sc_exp(a):
    y = jnp.clip(a * 1.4426950408889634, -126.0, 126.0)
    n = (y + 512.5).astype(jnp.int32) - 512
    t = (y - n.astype(jnp.float32)) * 0.6931471805599453
    p = 1.0 + t * (1.0 + t * (0.5 + t * (
        0.16666666666666666 + t * (0.041666666666666664 + t * 0.008333333333333333))))
    scale = lax.bitcast_convert_type(
        lax.shift_left(n + 127, 23), jnp.float32)
    return p * scale


def _zero_spmem(acc, zbuf, s, row_w):
    zv = jnp.zeros((LANES,), jnp.float32)

    def zrow(r, carry):
        for v in range(row_w // LANES):
            zbuf[r, pl.ds(LANES * v, LANES)] = zv
        return carry

    lax.fori_loop(0, RU, zrow, 0)

    def unit(j, carry):
        pltpu.sync_copy(zbuf, acc.at[pl.ds((s + NS * j) * RU, RU)])
        return carry

    lax.fori_loop(0, _n_units(s), unit, 0)


_B0 = 80


@functools.partial(
    pl.kernel,
    out_type=jax.ShapeDtypeStruct((NC, N, 32), jnp.float32),
    mesh=_mesh,
    scratch_types=[
        pltpu.VMEM((_B0,), jnp.int32),
        pltpu.VMEM((_B0, DE), jnp.float32),
        pltpu.VMEM((_B0, 32), jnp.float32),
        pltpu.VMEM((RU, 32), jnp.float32),
        pltpu.VMEM_SHARED((N, 32), jnp.float32),
    ],
    compiler_params=pltpu.CompilerParams(use_tc_tiling_on_sc=False),
)
def _sc_hist(tgt_hbm, attr_hbm, out_hbm, tgt_v, attr_v, row_v, zbuf, acc):
    c = lax.axis_index("c")
    s = lax.axis_index("s")
    wid = s * NC + c
    _zero_spmem(acc, zbuf, s, 32)
    plsc.subcore_barrier()

    iot = lax.iota(jnp.int32, LANES)
    cntvec = jnp.where(iot == 0, 1.0, 0.0).astype(jnp.float32)

    def chunk(i, carry):
        base = wid * EPW + i * _B0
        pltpu.sync_copy(tgt_hbm.at[pl.ds(base, _B0)], tgt_v)
        pltpu.sync_copy(attr_hbm.at[pl.ds(base, _B0)], attr_v)

        def edge(e, cc):
            row_v[e, pl.ds(0, LANES)] = attr_v[e, pl.ds(0, LANES)]
            row_v[e, pl.ds(LANES, LANES)] = cntvec
            return cc

        lax.fori_loop(0, _B0, edge, 0)
        pltpu.sync_copy(row_v, acc.at[tgt_v], add=True)
        return carry

    lax.fori_loop(0, EPW // _B0, chunk, 0)
    plsc.subcore_barrier()

    def unit(j, carry):
        r0 = (s + NS * j) * RU
        pltpu.sync_copy(acc.at[pl.ds(r0, RU)], out_hbm.at[c, pl.ds(r0, RU)])
        return carry

    lax.fori_loop(0, _n_units(s), unit, 0)


_BE = 80
_NCH = EPW // _BE


def _make_gat_pass(H):
    @functools.partial(
        pl.kernel,
        out_type=jax.ShapeDtypeStruct((H, NC, N, _ROW), jnp.float32),
        mesh=_mesh,
        scratch_types=[
            pltpu.VMEM((EPW,), jnp.int32),
            pltpu.VMEM((EPW,), jnp.int32),
            [pltpu.VMEM((_BE, HC), jnp.float32) for _ in range(2)],
            [pltpu.VMEM((_BE, HC), jnp.float32) for _ in range(2)],
            [pltpu.VMEM((_BE, HC), jnp.float32) for _ in range(2)],
            [pltpu.VMEM((_BE, _ROW), jnp.float32) for _ in range(2)],
            [pltpu.VMEM((_BE,), jnp.int32) for _ in range(2)],
            pltpu.VMEM((RU, _ROW), jnp.float32),
            pltpu.VMEM((H * HC,), jnp.float32),
            pltpu.VMEM_SHARED((N, _ROW), jnp.float32),
            [pltpu.SemaphoreType.DMA for _ in range(2)],
            [pltpu.SemaphoreType.DMA for _ in range(2)],
        ],
        compiler_params=pltpu.CompilerParams(use_tc_tiling_on_sc=False),
    )
    def gat_pass(src_hbm, tgt_hbm, xl_hbm, xr_hbm, ea_hbm, att_hbm, out_hbm,
                 src_all, tgt_all, xl_v, xr_v, ea_v, row_v, tgt_v, zbuf,
                 att_v, acc, gsem, ssem):
        c = lax.axis_index("c")
        s = lax.axis_index("s")
        wid = s * NC + c
        ebase = wid * EPW
        pltpu.sync_copy(att_hbm, att_v)
        pltpu.sync_copy(src_hbm.at[pl.ds(ebase, EPW)], src_all)
        pltpu.sync_copy(tgt_hbm.at[pl.ds(ebase, EPW)], tgt_all)
        iot = lax.iota(jnp.int32, LANES)

        for h in range(H):
            _zero_spmem(acc, zbuf, s, _ROW)
            plsc.subcore_barrier()
            atts = [att_v[pl.ds(h * HC + LANES * v, LANES)]
                    for v in range(HC // LANES)]

            def issue_gathers(ci, b):
                pltpu.async_copy(
                    xl_hbm.at[h].at[src_all.at[pl.ds(ci * _BE, _BE)]],
                    xl_v[b], gsem[b])
                pltpu.async_copy(
                    xr_hbm.at[h].at[tgt_all.at[pl.ds(ci * _BE, _BE)]],
                    xr_v[b], gsem[b])
                pltpu.async_copy(
                    ea_hbm.at[h].at[pl.ds(ebase + ci * _BE, _BE)],
                    ea_v[b], gsem[b])

            def wait_gathers(b):
                for _ in range(3):
                    pltpu.make_async_copy(
                        ea_hbm.at[h].at[pl.ds(0, _BE)], xl_v[b],
                        gsem[b]).wait()

            def wait_scatter(b):
                pltpu.make_async_copy(
                    row_v[b], acc.at[tgt_v[b]], ssem[b]).wait()

            def compute_scatter(ci, b):
                xlb, xrb, eab, rowb, tgtb = (xl_v[b], xr_v[b], ea_v[b],
                                             row_v[b], tgt_v[b])

                def edge(e, cc):
                    xls = [xlb[e, pl.ds(LANES * v, LANES)] for v in range(4)]
                    p = None
                    for v in range(4):
                        z = xls[v] + xrb[e, pl.ds(LANES * v, LANES)] \
                            + eab[e, pl.ds(LANES * v, LANES)]
                        z = jnp.where(z > 0.0, z, 0.2 * z)
                        t = z * atts[v]
                        p = t if p is None else p + t
                    exv = _sc_exp(_hsum(p, iot))
                    for v in range(4):
                        rowb[e, pl.ds(LANES * v, LANES)] = xls[v] * exv
                    rowb[e, pl.ds(HC, LANES)] = jnp.where(iot == 0, exv, 0.0)
                    return cc

                lax.fori_loop(0, _BE, edge, 0)
                for v in range(_BE // LANES):
                    tgtb[pl.ds(LANES * v, LANES)] = \
                        tgt_all[pl.ds(ci * _BE + LANES * v, LANES)]
                pltpu.async_copy(rowb, acc.at[tgtb], ssem[b], add=True)

            issue_gathers(0, 0)

            def pair(j, carry):
                c0 = 2 * j
                issue_gathers(c0 + 1, 1)
                wait_gathers(0)

                @pl.when(j > 0)
                def _():
                    wait_scatter(0)

                compute_scatter(c0, 0)
                issue_gathers(c0 + 2, 0)
                wait_gathers(1)

                @pl.when(j > 0)
                def _():
                    wait_scatter(1)

                compute_scatter(c0 + 1, 1)
                return carry

            lax.fori_loop(0, (_NCH - 1) // 2, pair, 0)
            wait_gathers(0)
            wait_scatter(0)
            compute_scatter(_NCH - 1, 0)
            wait_scatter(0)
            wait_scatter(1)
            plsc.subcore_barrier()

            def unit(j, carry):
                r0 = (s + NS * j) * RU
                pltpu.sync_copy(acc.at[pl.ds(r0, RU)],
                                out_hbm.at[h].at[c].at[pl.ds(r0, RU)])
                return carry

            lax.fori_loop(0, _n_units(s), unit, 0)

    return gat_pass


_sc_gat3 = _make_gat_pass(3)
_sc_gat1 = _make_gat_pass(1)


_NBLK = 1000
_EBLK = 4000


def _tc_nodes_body(x_ref, wl_ref, wr_ref, xl_ref, xr_ref):
    x = x_ref[...].astype(jnp.bfloat16)
    xl_ref[...] = jnp.dot(x, wl_ref[...].astype(jnp.bfloat16),
                          preferred_element_type=jnp.float32).reshape(
                              -1, 3, HC).swapaxes(0, 1)
    xr_ref[...] = jnp.dot(x, wr_ref[...].astype(jnp.bfloat16),
                          preferred_element_type=jnp.float32).reshape(
                              -1, 3, HC).swapaxes(0, 1)


def _tc_ea_body(attr_ref, we1_ref, we2_ref, ea1_ref, ea2_ref):
    a = attr_ref[...].astype(jnp.bfloat16)
    ea1_ref[...] = jnp.dot(a, we1_ref[...].astype(jnp.bfloat16),
                           preferred_element_type=jnp.float32).reshape(
                               -1, 3, HC).swapaxes(0, 1)
    ea2_ref[...] = jnp.dot(a, we2_ref[...].astype(jnp.bfloat16), preferred_element_type=jnp.float32)


def _tc_asm1_body(acc_ref, attr_ref, xl_ref, xr_ref, we1_ref, att1_ref, b1_ref,
                  wl2_ref, wr2_ref, we2_ref, xl2_ref, xr2_ref, sea2_ref):
    attr = attr_ref[0] + attr_ref[1]
    cnt = jnp.clip(attr[:, DE:DE + 1], 1.0, None)
    loop_attr = attr[:, :DE] / cnt
    sea = jnp.dot(loop_attr.astype(jnp.bfloat16), we1_ref[...].astype(jnp.bfloat16), preferred_element_type=jnp.float32)
    outs = []
    for h in range(3):
        xlh = xl_ref[h]
        zh = xlh + xr_ref[h] + sea[:, HC * h:HC * (h + 1)]
        zh = jnp.where(zh > 0.0, zh, 0.2 * zh)
        alpha = jnp.sum(zh * att1_ref[h][None, :], axis=1, keepdims=True)
        ex = jnp.exp(alpha)
        accs = acc_ref[h, 0] + acc_ref[h, 1]
        num = accs[:, :HC] + ex * xlh
        den = accs[:, HC:HC + 1] + ex
        outs.append(num / den)
    hh = jnp.concatenate(outs, axis=1) + b1_ref[...]
    hh = jnp.where(hh > 0.0, hh, jnp.exp(jnp.minimum(hh, 0.0)) - 1.0)
    xl2_ref[...] = jnp.dot(hh.astype(jnp.bfloat16), wl2_ref[...].astype(jnp.bfloat16), preferred_element_type=jnp.float32)
    xr2_ref[...] = jnp.dot(hh.astype(jnp.bfloat16), wr2_ref[...].astype(jnp.bfloat16), preferred_element_type=jnp.float32)
    sea2_ref[...] = jnp.dot(loop_attr.astype(jnp.bfloat16), we2_ref[...].astype(jnp.bfloat16),
                            preferred_element_type=jnp.float32)


def _tc_final_body(acc_ref, xl2_ref, xr2_ref, sea2_ref, att2_ref, b2_ref,
                   batch_ref, wih_ref, whh_ref, bih_ref, bhh_ref,
                   l1w_ref, l1b_ref, l2w_ref, l2b_ref, out_ref, fr_ref):
    acc = acc_ref[0, 0] + acc_ref[0, 1]
    xl2 = xl2_ref[...]
    z = xl2 + xr2_ref[...] + sea2_ref[...]
    z = jnp.where(z > 0.0, z, 0.2 * z)
    alpha = jnp.sum(z * att2_ref[...], axis=1, keepdims=True)
    ex = jnp.exp(alpha)
    num = acc[:, :HC] + ex * xl2
    den = acc[:, HC:HC + 1] + ex
    h2 = num / den + b2_ref[...]
    h2 = jnp.where(h2 > 0.0, h2, jnp.exp(jnp.minimum(h2, 0.0)) - 1.0)

    gids = lax.broadcasted_iota(jnp.int32, (NG, N), 0)
    mask = (gids == batch_ref[...]).astype(jnp.float32)
    gsum = jnp.dot(mask, h2, preferred_element_type=jnp.float32, precision=lax.Precision.HIGHEST)
    gcnt = jnp.clip(jnp.sum(mask, axis=1, keepdims=True), 1.0, None)
    fr_ref[...] = gsum / gcnt

    wih = wih_ref[...].astype(jnp.bfloat16)
    whh = whh_ref[...].astype(jnp.bfloat16)
    bsum = bih_ref[...] + bhh_ref[...]

    def step(t, carry):
        hh, cc = carry
        xt = fr_ref[pl.ds(t, 1), :]
        g = jnp.dot(xt.astype(jnp.bfloat16), wih, preferred_element_type=jnp.float32) \
            + jnp.dot(hh.astype(jnp.bfloat16), whh, preferred_element_type=jnp.float32) + bsum
        gi = jax.nn.sigmoid(g[:, :HC])
        gf = jax.nn.sigmoid(g[:, HC:2 * HC])
        gg = jnp.tanh(g[:, 2 * HC:3 * HC])
        go = jax.nn.sigmoid(g[:, 3 * HC:])
        c2 = gf * cc + gi * gg
        return (go * jnp.tanh(c2), c2)

    zed = jnp.zeros((1, HC), jnp.float32)
    hlast, _ = lax.fori_loop(0, NG, step, (zed, zed))
    o = jnp.dot(hlast.astype(jnp.bfloat16), l1w_ref[...].astype(jnp.bfloat16), preferred_element_type=jnp.float32) \
        + l1b_ref[...]
    o = jnp.maximum(o, 0.0)
    out_ref[...] = jnp.dot(o.astype(jnp.bfloat16), l2w_ref[...].astype(jnp.bfloat16),
                           preferred_element_type=jnp.float32) + l2b_ref[...]


def kernel(x, edge_index, edge_attr, batch, num_graphs, W_l1, W_r1, W_e1, att1,
           b1, W_l2, W_r2, W_e2, att2, b2, W_ih, W_hh, b_ih, b_hh, lin1_W,
           lin1_b, lin2_W, lin2_b):
    f32 = jnp.float32
    src = edge_index[0]
    tgt = edge_index[1]

    xl1, xr1 = pl.pallas_call(
        _tc_nodes_body,
        grid=(N // _NBLK,),
        in_specs=[
            pl.BlockSpec((_NBLK, DF), lambda i: (i, 0)),
            pl.BlockSpec((DF, _D1), lambda i: (0, 0)),
            pl.BlockSpec((DF, _D1), lambda i: (0, 0)),
        ],
        out_specs=[
            pl.BlockSpec((3, _NBLK, HC), lambda i: (0, i, 0)),
            pl.BlockSpec((3, _NBLK, HC), lambda i: (0, i, 0)),
        ],
        out_shape=[
            jax.ShapeDtypeStruct((3, N, HC), f32),
            jax.ShapeDtypeStruct((3, N, HC), f32),
        ],
    )(x, W_l1, W_r1)

    ea1, ea2b = pl.pallas_call(
        _tc_ea_body,
        grid=(E // _EBLK,),
        in_specs=[
            pl.BlockSpec((_EBLK, DE), lambda i: (i, 0)),
            pl.BlockSpec((DE, _D1), lambda i: (0, 0)),
            pl.BlockSpec((DE, HC), lambda i: (0, 0)),
        ],
        out_specs=[
            pl.BlockSpec((3, _EBLK, HC), lambda i: (0, i, 0)),
            pl.BlockSpec((_EBLK, HC), lambda i: (i, 0)),
        ],
        out_shape=[
            jax.ShapeDtypeStruct((3, E, HC), f32),
            jax.ShapeDtypeStruct((E, HC), f32),
        ],
    )(edge_attr, W_e1, W_e2)

    attr_p = _sc_hist(tgt, edge_attr)
    acc1_p = _sc_gat3(src, tgt, xl1, xr1, ea1, att1.reshape(_D1))

    xl2, xr2, sea2 = pl.pallas_call(
        _tc_asm1_body,
        grid=(N // _NBLK,),
        in_specs=[
            pl.BlockSpec((3, NC, _NBLK, _ROW), lambda i: (0, 0, i, 0)),
            pl.BlockSpec((NC, _NBLK, 32), lambda i: (0, i, 0)),
            pl.BlockSpec((3, _NBLK, HC), lambda i: (0, i, 0)),
            pl.BlockSpec((3, _NBLK, HC), lambda i: (0, i, 0)),
            pl.BlockSpec((DE, _D1), lambda i: (0, 0)),
            pl.BlockSpec((3, HC), lambda i: (0, 0)),
            pl.BlockSpec((1, _D1), lambda i: (0, 0)),
            pl.BlockSpec((_D1, HC), lambda i: (0, 0)),
            pl.BlockSpec((_D1, HC), lambda i: (0, 0)),
            pl.BlockSpec((DE, HC), lambda i: (0, 0)),
        ],
        out_specs=[
            pl.BlockSpec((_NBLK, HC), lambda i: (i, 0)),
            pl.BlockSpec((_NBLK, HC), lambda i: (i, 0)),
            pl.BlockSpec((_NBLK, HC), lambda i: (i, 0)),
        ],
        out_shape=[
            jax.ShapeDtypeStruct((N, HC), f32),
            jax.ShapeDtypeStruct((N, HC), f32),
            jax.ShapeDtypeStruct((N, HC), f32),
        ],
    )(acc1_p, attr_p, xl1, xr1, W_e1, att1, b1.reshape(1, _D1),
      W_l2, W_r2, W_e2)

    acc2_p = _sc_gat1(src, tgt, xl2.reshape(1, N, HC), xr2.reshape(1, N, HC),
                      ea2b.reshape(1, E, HC), att2.reshape(HC))

    out = pl.pallas_call(
        _tc_final_body,
        out_shape=jax.ShapeDtypeStruct((1, 1), f32),
        scratch_shapes=[pltpu.VMEM((NG, HC), f32)],
    )(acc2_p, xl2, xr2, sea2, att2, b2.reshape(1, HC), batch.reshape(1, N),
      W_ih.T, W_hh.T, b_ih.reshape(1, 4 * HC), b_hh.reshape(1, 4 * HC),
      lin1_W, lin1_b.reshape(1, HC // 2), lin2_W, lin2_b.reshape(1, 1))
    return out


# speedup vs baseline: 20.1586x; 1.2512x over previous
import functools

import jax
import jax.numpy as jnp
from jax import lax
from jax.experimental import pallas as pl
from jax.experimental.pallas import tpu as pltpu
from jax.experimental.pallas import tpu_sc as plsc

N = 10000
E = 320000
DF = 128
DE = 16
HC = 64
NG = 20

NC = 2
NS = 16
LANES = 16
NW = NC * NS
EPW = E // NW
RU = 80
NU = N // RU
_D1 = 3 * HC
_ROW = HC + LANES

_mesh = plsc.VectorSubcoreMesh(core_axis_name="c", subcore_axis_name="s")


def _n_units(s):
    return (NU - s + NS - 1) // NS


_GDN = lax.GatherDimensionNumbers(
    offset_dims=(), collapsed_slice_dims=(0,), start_index_map=(0,))


def _lperm(v, idx):
    return lax.gather(v, idx[:, None], dimension_numbers=_GDN, slice_sizes=(1,),
                      mode=lax.GatherScatterMode.PROMISE_IN_BOUNDS)


def _hsum(v, iot):
    for sh in (8, 4, 2, 1):
        v = v + _lperm(v, iot ^ sh)
    return v


def _sc_exp(a):
    y = jnp.clip(a * 1.4426950408889634, -126.0, 126.0)
    n = (y + 512.5).astype(jnp.int32) - 512
    t = (y - n.astype(jnp.float32)) * 0.6931471805599453
    p = 1.0 + t * (1.0 + t * (0.5 + t * (
        0.16666666666666666 + t * (0.041666666666666664 + t * 0.008333333333333333))))
    scale = lax.bitcast_convert_type(
        lax.shift_left(n + 127, 23), jnp.float32)
    return p * scale


def _zero_spmem(acc, zbuf, s, row_w):
    zv = jnp.zeros((LANES,), jnp.float32)

    def zrow(r, carry):
        for v in range(row_w // LANES):
            zbuf[r, pl.ds(LANES * v, LANES)] = zv
        return carry

    lax.fori_loop(0, RU, zrow, 0)

    def unit(j, carry):
        pltpu.sync_copy(zbuf, acc.at[pl.ds((s + NS * j) * RU, RU)])
        return carry

    lax.fori_loop(0, _n_units(s), unit, 0)


_B0 = 80


@functools.partial(
    pl.kernel,
    out_type=jax.ShapeDtypeStruct((NC, N, 32), jnp.float32),
    mesh=_mesh,
    scratch_types=[
        pltpu.VMEM((_B0,), jnp.int32),
        pltpu.VMEM((_B0, DE), jnp.float32),
        pltpu.VMEM((_B0, 32), jnp.float32),
        pltpu.VMEM((RU, 32), jnp.float32),
        pltpu.VMEM_SHARED((N, 32), jnp.float32),
    ],
    compiler_params=pltpu.CompilerParams(use_tc_tiling_on_sc=False),
)
def _sc_hist(tgt_hbm, attr_hbm, out_hbm, tgt_v, attr_v, row_v, zbuf, acc):
    c = lax.axis_index("c")
    s = lax.axis_index("s")
    wid = s * NC + c
    _zero_spmem(acc, zbuf, s, 32)
    plsc.subcore_barrier()

    iot = lax.iota(jnp.int32, LANES)
    cntvec = jnp.where(iot == 0, 1.0, 0.0).astype(jnp.float32)

    def chunk(i, carry):
        base = wid * EPW + i * _B0
        pltpu.sync_copy(tgt_hbm.at[pl.ds(base, _B0)], tgt_v)
        pltpu.sync_copy(attr_hbm.at[pl.ds(base, _B0)], attr_v)

        def edge(e, cc):
            row_v[e, pl.ds(0, LANES)] = attr_v[e, pl.ds(0, LANES)]
            row_v[e, pl.ds(LANES, LANES)] = cntvec
            return cc

        lax.fori_loop(0, _B0, edge, 0)
        pltpu.sync_copy(row_v, acc.at[tgt_v], add=True)
        return carry

    lax.fori_loop(0, EPW // _B0, chunk, 0)
    plsc.subcore_barrier()

    def unit(j, carry):
        r0 = (s + NS * j) * RU
        pltpu.sync_copy(acc.at[pl.ds(r0, RU)], out_hbm.at[c, pl.ds(r0, RU)])
        return carry

    lax.fori_loop(0, _n_units(s), unit, 0)


_BE = 80
_NCH = EPW // _BE


def _make_gat_pass(H):
    @functools.partial(
        pl.kernel,
        out_type=jax.ShapeDtypeStruct((H, NC, N, _ROW), jnp.float32),
        mesh=_mesh,
        scratch_types=[
            pltpu.VMEM((EPW,), jnp.int32),
            pltpu.VMEM((EPW,), jnp.int32),
            [pltpu.VMEM((_BE, HC), jnp.float32) for _ in range(2)],
            [pltpu.VMEM((_BE, HC), jnp.float32) for _ in range(2)],
            [pltpu.VMEM((_BE, HC), jnp.float32) for _ in range(2)],
            [pltpu.VMEM((_BE, _ROW), jnp.float32) for _ in range(2)],
            [pltpu.VMEM((_BE,), jnp.int32) for _ in range(2)],
            pltpu.VMEM((RU, _ROW), jnp.float32),
            pltpu.VMEM((H * HC,), jnp.float32),
            pltpu.VMEM_SHARED((N, _ROW), jnp.float32),
            [pltpu.SemaphoreType.DMA for _ in range(2)],
            [pltpu.SemaphoreType.DMA for _ in range(2)],
        ],
        compiler_params=pltpu.CompilerParams(use_tc_tiling_on_sc=False),
    )
    def gat_pass(src_hbm, tgt_hbm, xl_hbm, xr_hbm, ea_hbm, att_hbm, out_hbm,
                 src_all, tgt_all, xl_v, xr_v, ea_v, row_v, tgt_v, zbuf,
                 att_v, acc, gsem, ssem):
        c = lax.axis_index("c")
        s = lax.axis_index("s")
        wid = s * NC + c
        ebase = wid * EPW
        pltpu.sync_copy(att_hbm, att_v)
        pltpu.sync_copy(src_hbm.at[pl.ds(ebase, EPW)], src_all)
        pltpu.sync_copy(tgt_hbm.at[pl.ds(ebase, EPW)], tgt_all)
        iot = lax.iota(jnp.int32, LANES)

        for h in range(H):
            _zero_spmem(acc, zbuf, s, _ROW)
            plsc.subcore_barrier()
            atts = [att_v[pl.ds(h * HC + LANES * v, LANES)]
                    for v in range(HC // LANES)]

            def issue_gathers(ci, b):
                pltpu.async_copy(
                    xl_hbm.at[h].at[src_all.at[pl.ds(ci * _BE, _BE)]],
                    xl_v[b], gsem[b])
                pltpu.async_copy(
                    xr_hbm.at[h].at[tgt_all.at[pl.ds(ci * _BE, _BE)]],
                    xr_v[b], gsem[b])
                pltpu.async_copy(
                    ea_hbm.at[h].at[pl.ds(ebase + ci * _BE, _BE)],
                    ea_v[b], gsem[b])

            def wait_gathers(b):
                for _ in range(3):
                    pltpu.make_async_copy(
                        ea_hbm.at[h].at[pl.ds(0, _BE)], xl_v[b],
                        gsem[b]).wait()

            def wait_scatter(b):
                pltpu.make_async_copy(
                    row_v[b], acc.at[tgt_v[b]], ssem[b]).wait()

            def compute_scatter(ci, b):
                xlb, xrb, eab, rowb, tgtb = (xl_v[b], xr_v[b], ea_v[b],
                                             row_v[b], tgt_v[b])

                def edge(e2, cc):
                    for e in (2 * e2, 2 * e2 + 1):
                        xls = [xlb[e, pl.ds(LANES * v, LANES)]
                               for v in range(4)]
                        p = None
                        for v in range(4):
                            z = xls[v] + xrb[e, pl.ds(LANES * v, LANES)] \
                                + eab[e, pl.ds(LANES * v, LANES)]
                            z = jnp.where(z > 0.0, z, 0.2 * z)
                            t = z * atts[v]
                            p = t if p is None else p + t
                        exv = jnp.exp(_hsum(p, iot))
                        for v in range(4):
                            rowb[e, pl.ds(LANES * v, LANES)] = xls[v] * exv
                        rowb[e, pl.ds(HC, LANES)] = jnp.where(iot == 0, exv,
                                                              0.0)
                    return cc

                lax.fori_loop(0, _BE // 2, edge, 0)
                for v in range(_BE // LANES):
                    tgtb[pl.ds(LANES * v, LANES)] = \
                        tgt_all[pl.ds(ci * _BE + LANES * v, LANES)]
                pltpu.async_copy(rowb, acc.at[tgtb], ssem[b], add=True)

            issue_gathers(0, 0)

            def pair(j, carry):
                c0 = 2 * j
                issue_gathers(c0 + 1, 1)
                wait_gathers(0)

                @pl.when(j > 0)
                def _():
                    wait_scatter(0)

                compute_scatter(c0, 0)
                issue_gathers(c0 + 2, 0)
                wait_gathers(1)

                @pl.when(j > 0)
                def _():
                    wait_scatter(1)

                compute_scatter(c0 + 1, 1)
                return carry

            lax.fori_loop(0, (_NCH - 1) // 2, pair, 0)
            wait_gathers(0)
            wait_scatter(0)
            compute_scatter(_NCH - 1, 0)
            wait_scatter(0)
            wait_scatter(1)
            plsc.subcore_barrier()

            def unit(j, carry):
                r0 = (s + NS * j) * RU
                pltpu.sync_copy(acc.at[pl.ds(r0, RU)],
                                out_hbm.at[h].at[c].at[pl.ds(r0, RU)])
                return carry

            lax.fori_loop(0, _n_units(s), unit, 0)

    return gat_pass


_sc_gat3 = _make_gat_pass(3)
_sc_gat1 = _make_gat_pass(1)


_NBLK = 1000
_EBLK = 4000


def _tc_nodes_body(x_ref, wl_ref, wr_ref, xl_ref, xr_ref):
    x = x_ref[...].astype(jnp.bfloat16)
    xl_ref[...] = jnp.dot(x, wl_ref[...].astype(jnp.bfloat16),
                          preferred_element_type=jnp.float32).reshape(
                              -1, 3, HC).swapaxes(0, 1)
    xr_ref[...] = jnp.dot(x, wr_ref[...].astype(jnp.bfloat16),
                          preferred_element_type=jnp.float32).reshape(
                              -1, 3, HC).swapaxes(0, 1)


def _tc_ea_body(attr_ref, we1_ref, we2_ref, ea1_ref, ea2_ref):
    a = attr_ref[...].astype(jnp.bfloat16)
    ea1_ref[...] = jnp.dot(a, we1_ref[...].astype(jnp.bfloat16),
                           preferred_element_type=jnp.float32).reshape(
                               -1, 3, HC).swapaxes(0, 1)
    ea2_ref[...] = jnp.dot(a, we2_ref[...].astype(jnp.bfloat16), preferred_element_type=jnp.float32)


def _tc_asm1_body(acc_ref, attr_ref, xl_ref, xr_ref, we1_ref, att1_ref, b1_ref,
                  wl2_ref, wr2_ref, we2_ref, xl2_ref, xr2_ref, sea2_ref):
    attr = attr_ref[0] + attr_ref[1]
    cnt = jnp.clip(attr[:, DE:DE + 1], 1.0, None)
    loop_attr = attr[:, :DE] / cnt
    sea = jnp.dot(loop_attr.astype(jnp.bfloat16), we1_ref[...].astype(jnp.bfloat16), preferred_element_type=jnp.float32)
    outs = []
    for h in range(3):
        xlh = xl_ref[h]
        zh = xlh + xr_ref[h] + sea[:, HC * h:HC * (h + 1)]
        zh = jnp.where(zh > 0.0, zh, 0.2 * zh)
        alpha = jnp.sum(zh * att1_ref[h][None, :], axis=1, keepdims=True)
        ex = jnp.exp(alpha)
        accs = acc_ref[h, 0] + acc_ref[h, 1]
        num = accs[:, :HC] + ex * xlh
        den = accs[:, HC:HC + 1] + ex
        outs.append(num / den)
    hh = jnp.concatenate(outs, axis=1) + b1_ref[...]
    hh = jnp.where(hh > 0.0, hh, jnp.exp(jnp.minimum(hh, 0.0)) - 1.0)
    xl2_ref[...] = jnp.dot(hh.astype(jnp.bfloat16), wl2_ref[...].astype(jnp.bfloat16), preferred_element_type=jnp.float32)
    xr2_ref[...] = jnp.dot(hh.astype(jnp.bfloat16), wr2_ref[...].astype(jnp.bfloat16), preferred_element_type=jnp.float32)
    sea2_ref[...] = jnp.dot(loop_attr.astype(jnp.bfloat16), we2_ref[...].astype(jnp.bfloat16),
                            preferred_element_type=jnp.float32)


def _tc_final_body(acc_ref, xl2_ref, xr2_ref, sea2_ref, att2_ref, b2_ref,
                   batch_ref, wih_ref, whh_ref, bih_ref, bhh_ref,
                   l1w_ref, l1b_ref, l2w_ref, l2b_ref, out_ref, fr_ref):
    acc = acc_ref[0, 0] + acc_ref[0, 1]
    xl2 = xl2_ref[...]
    z = xl2 + xr2_ref[...] + sea2_ref[...]
    z = jnp.where(z > 0.0, z, 0.2 * z)
    alpha = jnp.sum(z * att2_ref[...], axis=1, keepdims=True)
    ex = jnp.exp(alpha)
    num = acc[:, :HC] + ex * xl2
    den = acc[:, HC:HC + 1] + ex
    h2 = num / den + b2_ref[...]
    h2 = jnp.where(h2 > 0.0, h2, jnp.exp(jnp.minimum(h2, 0.0)) - 1.0)

    gids = lax.broadcasted_iota(jnp.int32, (NG, N), 0)
    mask = (gids == batch_ref[...]).astype(jnp.float32)
    gsum = jnp.dot(mask, h2, preferred_element_type=jnp.float32, precision=lax.Precision.HIGHEST)
    gcnt = jnp.clip(jnp.sum(mask, axis=1, keepdims=True), 1.0, None)
    fr_ref[...] = gsum / gcnt

    wih = wih_ref[...].astype(jnp.bfloat16)
    whh = whh_ref[...].astype(jnp.bfloat16)
    bsum = bih_ref[...] + bhh_ref[...]

    def step(t, carry):
        hh, cc = carry
        xt = fr_ref[pl.ds(t, 1), :]
        g = jnp.dot(xt.astype(jnp.bfloat16), wih, preferred_element_type=jnp.float32) \
            + jnp.dot(hh.astype(jnp.bfloat16), whh, preferred_element_type=jnp.float32) + bsum
        gi = jax.nn.sigmoid(g[:, :HC])
        gf = jax.nn.sigmoid(g[:, HC:2 * HC])
        gg = jnp.tanh(g[:, 2 * HC:3 * HC])
        go = jax.nn.sigmoid(g[:, 3 * HC:])
        c2 = gf * cc + gi * gg
        return (go * jnp.tanh(c2), c2)

    zed = jnp.zeros((1, HC), jnp.float32)
    hlast, _ = lax.fori_loop(0, NG, step, (zed, zed))
    o = jnp.dot(hlast.astype(jnp.bfloat16), l1w_ref[...].astype(jnp.bfloat16), preferred_element_type=jnp.float32) \
        + l1b_ref[...]
    o = jnp.maximum(o, 0.0)
    out_ref[...] = jnp.dot(o.astype(jnp.bfloat16), l2w_ref[...].astype(jnp.bfloat16),
                           preferred_element_type=jnp.float32) + l2b_ref[...]


def kernel(x, edge_index, edge_attr, batch, num_graphs, W_l1, W_r1, W_e1, att1,
           b1, W_l2, W_r2, W_e2, att2, b2, W_ih, W_hh, b_ih, b_hh, lin1_W,
           lin1_b, lin2_W, lin2_b):
    f32 = jnp.float32
    src = edge_index[0]
    tgt = edge_index[1]

    xl1, xr1 = pl.pallas_call(
        _tc_nodes_body,
        grid=(N // _NBLK,),
        in_specs=[
            pl.BlockSpec((_NBLK, DF), lambda i: (i, 0)),
            pl.BlockSpec((DF, _D1), lambda i: (0, 0)),
            pl.BlockSpec((DF, _D1), lambda i: (0, 0)),
        ],
        out_specs=[
            pl.BlockSpec((3, _NBLK, HC), lambda i: (0, i, 0)),
            pl.BlockSpec((3, _NBLK, HC), lambda i: (0, i, 0)),
        ],
        out_shape=[
            jax.ShapeDtypeStruct((3, N, HC), f32),
            jax.ShapeDtypeStruct((3, N, HC), f32),
        ],
    )(x, W_l1, W_r1)

    ea1, ea2b = pl.pallas_call(
        _tc_ea_body,
        grid=(E // _EBLK,),
        in_specs=[
            pl.BlockSpec((_EBLK, DE), lambda i: (i, 0)),
            pl.BlockSpec((DE, _D1), lambda i: (0, 0)),
            pl.BlockSpec((DE, HC), lambda i: (0, 0)),
        ],
        out_specs=[
            pl.BlockSpec((3, _EBLK, HC), lambda i: (0, i, 0)),
            pl.BlockSpec((_EBLK, HC), lambda i: (i, 0)),
        ],
        out_shape=[
            jax.ShapeDtypeStruct((3, E, HC), f32),
            jax.ShapeDtypeStruct((E, HC), f32),
        ],
    )(edge_attr, W_e1, W_e2)

    attr_p = _sc_hist(tgt, edge_attr)
    acc1_p = _sc_gat3(src, tgt, xl1, xr1, ea1, att1.reshape(_D1))

    xl2, xr2, sea2 = pl.pallas_call(
        _tc_asm1_body,
        grid=(N // _NBLK,),
        in_specs=[
            pl.BlockSpec((3, NC, _NBLK, _ROW), lambda i: (0, 0, i, 0)),
            pl.BlockSpec((NC, _NBLK, 32), lambda i: (0, i, 0)),
            pl.BlockSpec((3, _NBLK, HC), lambda i: (0, i, 0)),
            pl.BlockSpec((3, _NBLK, HC), lambda i: (0, i, 0)),
            pl.BlockSpec((DE, _D1), lambda i: (0, 0)),
            pl.BlockSpec((3, HC), lambda i: (0, 0)),
            pl.BlockSpec((1, _D1), lambda i: (0, 0)),
            pl.BlockSpec((_D1, HC), lambda i: (0, 0)),
            pl.BlockSpec((_D1, HC), lambda i: (0, 0)),
            pl.BlockSpec((DE, HC), lambda i: (0, 0)),
        ],
        out_specs=[
            pl.BlockSpec((_NBLK, HC), lambda i: (i, 0)),
            pl.BlockSpec((_NBLK, HC), lambda i: (i, 0)),
            pl.BlockSpec((_NBLK, HC), lambda i: (i, 0)),
        ],
        out_shape=[
            jax.ShapeDtypeStruct((N, HC), f32),
            jax.ShapeDtypeStruct((N, HC), f32),
            jax.ShapeDtypeStruct((N, HC), f32),
        ],
    )(acc1_p, attr_p, xl1, xr1, W_e1, att1, b1.reshape(1, _D1),
      W_l2, W_r2, W_e2)

    acc2_p = _sc_gat1(src, tgt, xl2.reshape(1, N, HC), xr2.reshape(1, N, HC),
                      ea2b.reshape(1, E, HC), att2.reshape(HC))

    out = pl.pallas_call(
        _tc_final_body,
        out_shape=jax.ShapeDtypeStruct((1, 1), f32),
        scratch_shapes=[pltpu.VMEM((NG, HC), f32)],
    )(acc2_p, xl2, xr2, sea2, att2, b2.reshape(1, HC), batch.reshape(1, N),
      W_ih.T, W_hh.T, b_ih.reshape(1, 4 * HC), b_hh.reshape(1, 4 * HC),
      lin1_W, lin1_b.reshape(1, HC // 2), lin2_W, lin2_b.reshape(1, 1))
    return out
